# Initial kernel scaffold; baseline (speedup 1.0000x reference)
import jax
import jax.numpy as jnp
from jax.experimental import pallas as pl


def kernel(pos, x, params):
    raise NotImplementedError("write your pallas kernel here")


# speedup vs baseline: 3.5008x; 3.5008x over previous
import functools

import jax
import jax.numpy as jnp
from jax import lax
from jax.experimental import pallas as pl
from jax.experimental.pallas import tpu as pltpu
from jax.experimental.pallas import tpu_sc as plsc

N = 10000
K = 16
E = N * K
NW = 32
CHUNK = 128
E_PAD = 163840
CH_PER_W = E_PAD // (NW * CHUNK)
T = 400
NT = N // T
RT = 2000
NRT = N // RT
QT = 200
NPAD = 10240

f32 = jnp.float32
i32 = jnp.int32


def _lrelu(h):
    return jnp.where(h > 0, h, 0.2 * h)


def _knn_body(q_ref, pt_ref, out_ref, d_scr):
    q = q_ref[...]
    pt = pt_ref[...]
    qn = jnp.sum(q * q, axis=1, keepdims=True)
    pn = jnp.sum(pt * pt, axis=0, keepdims=True)
    mm = jnp.dot(q, pt, preferred_element_type=f32)
    d_scr[...] = qn + pn - 2.0 * mm

    col = lax.broadcasted_iota(i32, (QT, NPAD), 1)
    lane16 = lax.broadcasted_iota(i32, (QT, K), 1)

    def step(k, acc):
        d = d_scr[...]
        m = jnp.min(d, axis=1, keepdims=True)
        idx = jnp.min(jnp.where(d == m, col, i32(NPAD)), axis=1, keepdims=True)
        d_scr[...] = jnp.where(col == idx, f32(3e38), d)
        return jnp.where(lane16 == k, idx, acc)

    out_ref[...] = lax.fori_loop(0, K, step, jnp.zeros((QT, K), i32))


def _knn(pos8, posT8):
    return pl.pallas_call(
        _knn_body,
        grid=(N // QT,),
        in_specs=[
            pl.BlockSpec((QT, 8), lambda i: (i, 0)),
            pl.BlockSpec((8, NPAD), lambda i: (0, 0)),
        ],
        out_specs=pl.BlockSpec((QT, K), lambda i: (i, 0)),
        out_shape=jax.ShapeDtypeStruct((N, K), i32),
        scratch_shapes=[pltpu.VMEM((QT, NPAD), f32)],
    )(pos8, posT8)


def _linstat(heads):
    in_specs, args, douts = [], [], []
    for (x, sc, sh, w, b) in heads:
        din, dout = w.shape
        douts.append(dout)
        in_specs.append(pl.BlockSpec((RT, din), lambda i: (i, 0)))
        args.append(x)
        if sc is not None:
            in_specs.append(pl.BlockSpec((1, din), lambda i: (0, 0)))
            args.append(sc)
            in_specs.append(pl.BlockSpec((1, din), lambda i: (0, 0)))
            args.append(sh)
        in_specs.append(pl.BlockSpec((din, dout), lambda i: (0, 0)))
        args.append(w)
        in_specs.append(pl.BlockSpec((1, dout), lambda i: (0, 0)))
        args.append(b)

    out_specs, out_shapes = [], []
    for dout in douts:
        out_specs += [
            pl.BlockSpec((RT, dout), lambda i: (i, 0)),
            pl.BlockSpec((1, dout), lambda i: (0, 0)),
            pl.BlockSpec((1, dout), lambda i: (0, 0)),
        ]
        out_shapes += [
            jax.ShapeDtypeStruct((N, dout), f32),
            jax.ShapeDtypeStruct((1, dout), f32),
            jax.ShapeDtypeStruct((1, dout), f32),
        ]

    has_pre = [h[1] is not None for h in heads]

    def body(*refs):
        i = pl.program_id(0)
        pos = 0
        ins = []
        for hp in has_pre:
            n_in = 5 if hp else 3
            ins.append(refs[pos:pos + n_in])
            pos += n_in
        outs = refs[pos:]
        for hi, hrefs in enumerate(ins):
            if has_pre[hi]:
                x_ref, sc_ref, sh_ref, w_ref, b_ref = hrefs
                x = _lrelu(x_ref[...] * sc_ref[...] + sh_ref[...])
            else:
                x_ref, w_ref, b_ref = hrefs
                x = x_ref[...]
            t = jnp.dot(x, w_ref[...], preferred_element_type=f32) + b_ref[...]
            t_ref, s_ref, q_ref = outs[3 * hi:3 * hi + 3]
            t_ref[...] = t

            @pl.when(i == 0)
            def _():
                s_ref[...] = jnp.zeros_like(s_ref)
                q_ref[...] = jnp.zeros_like(q_ref)

            s_ref[...] += jnp.sum(t, axis=0, keepdims=True)
            q_ref[...] += jnp.sum(t * t, axis=0, keepdims=True)

    flat = pl.pallas_call(
        body,
        grid=(NRT,),
        in_specs=in_specs,
        out_specs=out_specs,
        out_shape=out_shapes,
    )(*args)
    return [tuple(flat[3 * i:3 * i + 3]) for i in range(len(heads))]


def _bn_fold(s, q, n, g, be):
    m = s / n
    v = q / n - m * m
    scale = g[None, :] / jnp.sqrt(v + 1e-6)
    shift = be[None, :] - m * scale
    return scale, shift


def _s2b_body(tsc_ref, t1_ref, p8_ref, scs_ref, sch_ref, h1s_ref, h1h_ref,
              adg_ref, cdg_ref, bdg_ref, aen_ref, cen_ref, ben_ref,
              sc_out, ae1_out, u1_out, tab_out):
    sc_out[...] = tsc_ref[...] * scs_ref[...] + sch_ref[...]
    h1 = _lrelu(t1_ref[...] * h1s_ref[...] + h1h_ref[...])
    p8 = p8_ref[...]
    ae1_out[...] = jnp.dot(p8, aen_ref[...], preferred_element_type=f32) + ben_ref[...]
    u1_out[...] = jnp.dot(h1, adg_ref[...], preferred_element_type=f32) + bdg_ref[...]
    ce1 = jnp.dot(p8, cen_ref[...], preferred_element_type=f32)
    w1 = jnp.dot(h1, cdg_ref[...], preferred_element_type=f32)
    tab_out[...] = jnp.concatenate(
        [h1, ce1, w1, p8[:, 0:3], jnp.zeros((RT, 61), f32)], axis=1)


def _s2b(t_sc, t1, pos8, consts):
    vec = lambda c: pl.BlockSpec((1, c), lambda i: (0, 0))
    mat = lambda a, b: pl.BlockSpec((a, b), lambda i: (0, 0))
    return pl.pallas_call(
        _s2b_body,
        grid=(NRT,),
        in_specs=[
            pl.BlockSpec((RT, 128), lambda i: (i, 0)),
            pl.BlockSpec((RT, 16), lambda i: (i, 0)),
            pl.BlockSpec((RT, 8), lambda i: (i, 0)),
            vec(128), vec(128), vec(16), vec(16),
            mat(16, 32), mat(16, 32), vec(32),
            mat(8, 16), mat(8, 16), vec(16),
        ],
        out_specs=[
            pl.BlockSpec((RT, 128), lambda i: (i, 0)),
            pl.BlockSpec((RT, 16), lambda i: (i, 0)),
            pl.BlockSpec((RT, 32), lambda i: (i, 0)),
            pl.BlockSpec((RT, 128), lambda i: (i, 0)),
        ],
        out_shape=[
            jax.ShapeDtypeStruct((N, 128), f32),
            jax.ShapeDtypeStruct((N, 16), f32),
            jax.ShapeDtypeStruct((N, 32), f32),
            jax.ShapeDtypeStruct((N, 128), f32),
        ],
    )(t_sc, t1, pos8, *consts)


def _sc_gather_rows(table, idx3, rowlen):
    mesh = plsc.VectorSubcoreMesh(core_axis_name="c", subcore_axis_name="s")

    def body(idx_hbm, tab_hbm, out_hbm, idx_v, rows_v, sem):
        wid = lax.axis_index("s") * 2 + lax.axis_index("c")

        def chunk(c, carry):
            pltpu.sync_copy(idx_hbm.at[wid, c], idx_v)
            pltpu.async_copy(tab_hbm.at[idx_v], rows_v, sem).wait()
            pltpu.sync_copy(
                rows_v, out_hbm.at[pl.ds((wid * CH_PER_W + c) * CHUNK, CHUNK)])
            return carry

        lax.fori_loop(0, CH_PER_W, chunk, 0)

    return pl.kernel(
        body,
        out_type=jax.ShapeDtypeStruct((E_PAD, rowlen), f32),
        mesh=mesh,
        scratch_types=[
            pltpu.VMEM((CHUNK,), i32),
            pltpu.VMEM((CHUNK, rowlen), f32),
            pltpu.SemaphoreType.DMA,
        ],
    )(idx3, table)


def _edge_pre(g_ref, ae_ref, u_ref, p8_ref, wd_ref, c, rowlen):
    g3 = g_ref[...].reshape(T, K, rowlen)
    xj3 = g3[:, :, 0:c]
    cj3 = g3[:, :, c:2 * c]
    wj3 = g3[:, :, 2 * c:4 * c]
    pj3 = g3[:, :, 4 * c:4 * c + 3]
    pi3 = p8_ref[...][:, None, 0:3]
    pd3 = pj3 - pi3
    dist3 = jnp.sqrt(jnp.maximum(
        jnp.sum(pd3 * pd3, axis=2, keepdims=True), 1e-12))
    wd3 = wd_ref[...].reshape(1, 1, c)
    henc3 = ae_ref[...][:, None, :] + cj3 + dist3 * wd3
    hdg3 = u_ref[...][:, None, :] + wj3
    return xj3, henc3, hdg3


def _edge_stats(gath, ae, u, pos8, wd, c, rowlen):
    def body(g_ref, ae_ref, u_ref, p8_ref, wd_ref, se, qe, sd, qd):
        i = pl.program_id(0)
        _, henc3, hdg3 = _edge_pre(g_ref, ae_ref, u_ref, p8_ref, wd_ref, c, rowlen)

        @pl.when(i == 0)
        def _():
            se[...] = jnp.zeros_like(se)
            qe[...] = jnp.zeros_like(qe)
            sd[...] = jnp.zeros_like(sd)
            qd[...] = jnp.zeros_like(qd)

        se[...] += jnp.sum(jnp.sum(henc3, axis=1), axis=0, keepdims=True)
        qe[...] += jnp.sum(jnp.sum(henc3 * henc3, axis=1), axis=0, keepdims=True)
        sd[...] += jnp.sum(jnp.sum(hdg3, axis=1), axis=0, keepdims=True)
        qd[...] += jnp.sum(jnp.sum(hdg3 * hdg3, axis=1), axis=0, keepdims=True)

    return pl.pallas_call(
        body,
        grid=(NT,),
        in_specs=[
            pl.BlockSpec((K * T, rowlen), lambda i: (i, 0)),
            pl.BlockSpec((T, c), lambda i: (i, 0)),
            pl.BlockSpec((T, 2 * c), lambda i: (i, 0)),
            pl.BlockSpec((T, 8), lambda i: (i, 0)),
            pl.BlockSpec((1, c), lambda i: (0, 0)),
        ],
        out_specs=[pl.BlockSpec((1, c), lambda i: (0, 0)),
                   pl.BlockSpec((1, c), lambda i: (0, 0)),
                   pl.BlockSpec((1, 2 * c), lambda i: (0, 0)),
                   pl.BlockSpec((1, 2 * c), lambda i: (0, 0))],
        out_shape=[jax.ShapeDtypeStruct((1, c), f32),
                   jax.ShapeDtypeStruct((1, c), f32),
                   jax.ShapeDtypeStruct((1, 2 * c), f32),
                   jax.ShapeDtypeStruct((1, 2 * c), f32)],
    )(gath, ae, u, pos8, wd)


def _edge_aggr(gath, ae, u, pos8, wd, enc_sc, enc_sh, dg_sc, dg_sh, watt,
               c, rowlen, with_raw):
    def body(g_ref, ae_ref, u_ref, p8_ref, wd_ref, es_ref, eh_ref, ds_ref,
             dh_ref, w_ref, att_out, *maybe_raw):
        xj3, henc3, hdg3 = _edge_pre(g_ref, ae_ref, u_ref, p8_ref, wd_ref, c, rowlen)
        lse3 = _lrelu(henc3 * es_ref[...].reshape(1, 1, c)
                      + eh_ref[...].reshape(1, 1, c))
        dg3 = _lrelu(hdg3 * ds_ref[...].reshape(1, 1, 2 * c)
                     + dh_ref[...].reshape(1, 1, 2 * c))
        local3 = jnp.concatenate([dg3, xj3, lse3], axis=2)
        att2 = jnp.dot(local3.reshape(T * K, 4 * c), w_ref[...],
                       preferred_element_type=f32)
        att3 = att2.reshape(T, K, 4 * c)
        m3 = jnp.max(att3, axis=1, keepdims=True)
        e3 = jnp.exp(att3 - m3)
        ssum3 = jnp.sum(e3, axis=1, keepdims=True)
        scores3 = e3 / (ssum3 + 1e-16)
        att_out[...] = jnp.sum(scores3 * local3, axis=1)
        if with_raw:
            maybe_raw[0][...] = jnp.sum(local3, axis=1)

    out_specs = [pl.BlockSpec((T, 4 * c), lambda i: (i, 0))]
    out_shape = [jax.ShapeDtypeStruct((N, 4 * c), f32)]
    if with_raw:
        out_specs.append(pl.BlockSpec((T, 4 * c), lambda i: (i, 0)))
        out_shape.append(jax.ShapeDtypeStruct((N, 4 * c), f32))

    return pl.pallas_call(
        body,
        grid=(NT,),
        in_specs=[
            pl.BlockSpec((K * T, rowlen), lambda i: (i, 0)),
            pl.BlockSpec((T, c), lambda i: (i, 0)),
            pl.BlockSpec((T, 2 * c), lambda i: (i, 0)),
            pl.BlockSpec((T, 8), lambda i: (i, 0)),
            pl.BlockSpec((1, c), lambda i: (0, 0)),
            pl.BlockSpec((1, c), lambda i: (0, 0)),
            pl.BlockSpec((1, c), lambda i: (0, 0)),
            pl.BlockSpec((1, 2 * c), lambda i: (0, 0)),
            pl.BlockSpec((1, 2 * c), lambda i: (0, 0)),
            pl.BlockSpec((4 * c, 4 * c), lambda i: (0, 0)),
        ],
        out_specs=out_specs,
        out_shape=out_shape,
    )(gath, ae, u, pos8, wd, enc_sc, enc_sh, dg_sc, dg_sh, watt)


def _s6b_body(tp_ref, tr_ref, p8_ref, ps_ref, ph_ref, rs_ref, rh_ref,
              adg_ref, cdg_ref, bdg_ref, aen_ref, cen_ref, ben_ref,
              rec_out, ae2_out, u2_out, tab_out):
    h2 = _lrelu(tp_ref[...] * ps_ref[...] + ph_ref[...])
    rec_out[...] = _lrelu(tr_ref[...] * rs_ref[...] + rh_ref[...])
    p8 = p8_ref[...]
    ae2_out[...] = jnp.dot(p8, aen_ref[...], preferred_element_type=f32) + ben_ref[...]
    u2_out[...] = jnp.dot(h2, adg_ref[...], preferred_element_type=f32) + bdg_ref[...]
    ce2 = jnp.dot(p8, cen_ref[...], preferred_element_type=f32)
    w2 = jnp.dot(h2, cdg_ref[...], preferred_element_type=f32)
    tab_out[...] = jnp.concatenate(
        [h2, ce2, w2, p8[:, 0:3], jnp.zeros((RT, 125), f32)], axis=1)


def _s6b(t_post, t_raw, pos8, consts):
    vec = lambda c: pl.BlockSpec((1, c), lambda i: (0, 0))
    mat = lambda a, b: pl.BlockSpec((a, b), lambda i: (0, 0))
    return pl.pallas_call(
        _s6b_body,
        grid=(NRT,),
        in_specs=[
            pl.BlockSpec((RT, 32), lambda i: (i, 0)),
            pl.BlockSpec((RT, 128), lambda i: (i, 0)),
            pl.BlockSpec((RT, 8), lambda i: (i, 0)),
            vec(32), vec(32), vec(128), vec(128),
            mat(32, 64), mat(32, 64), vec(64),
            mat(8, 32), mat(8, 32), vec(32),
        ],
        out_specs=[
            pl.BlockSpec((RT, 128), lambda i: (i, 0)),
            pl.BlockSpec((RT, 32), lambda i: (i, 0)),
            pl.BlockSpec((RT, 64), lambda i: (i, 0)),
            pl.BlockSpec((RT, 256), lambda i: (i, 0)),
        ],
        out_shape=[
            jax.ShapeDtypeStruct((N, 128), f32),
            jax.ShapeDtypeStruct((N, 32), f32),
            jax.ShapeDtypeStruct((N, 64), f32),
            jax.ShapeDtypeStruct((N, 256), f32),
        ],
    )(t_post, t_raw, pos8, *consts)


def _s10c_body(tm_ref, sc_ref, s_ref, h_ref, out_ref):
    out_ref[...] = _lrelu(tm_ref[...] * s_ref[...] + h_ref[...] + sc_ref[...])


def _s10c(t_m, sc, scale, shift):
    return pl.pallas_call(
        _s10c_body,
        grid=(NRT,),
        in_specs=[
            pl.BlockSpec((RT, 128), lambda i: (i, 0)),
            pl.BlockSpec((RT, 128), lambda i: (i, 0)),
            pl.BlockSpec((1, 128), lambda i: (0, 0)),
            pl.BlockSpec((1, 128), lambda i: (0, 0)),
        ],
        out_specs=pl.BlockSpec((RT, 128), lambda i: (i, 0)),
        out_shape=jax.ShapeDtypeStruct((N, 128), f32),
    )(t_m, sc, scale, shift)


def _split3(w, c):
    a = w[0:c] - w[2 * c:3 * c]
    cc = w[c:2 * c] + w[2 * c:3 * c]
    return a, cc


def _split_enc(w):
    a = w[0:3] - w[6:9]
    cc = w[3:6] + w[6:9]
    pad = lambda m: jnp.concatenate([m, jnp.zeros((5, m.shape[1]), f32)], axis=0)
    return pad(a), pad(cc), w[9:10]


def kernel(pos, x, params):
    B = pos.shape[0]
    p = params
    pos2d = pos.reshape(N, 3)
    x2d = x.reshape(N, 128)
    pos8 = jnp.concatenate([pos2d, jnp.zeros((N, 5), f32)], axis=1)
    posT8 = jnp.concatenate(
        [pos2d.T, jnp.full((3, NPAD - N), 1e3, f32)], axis=1)
    posT8 = jnp.concatenate([posT8, jnp.zeros((5, NPAD), f32)], axis=0)

    nbr = _knn(pos8, posT8)
    idx_flat = nbr.reshape(E)
    idx3 = jnp.concatenate(
        [idx_flat, jnp.zeros((E_PAD - E,), i32)]).reshape(NW, CH_PER_W, CHUNK)

    row = lambda v: v[None, :]

    (t_sc, s_sc, q_sc), (t1, s1, q1) = _linstat([
        (x2d, None, None, p["sc"]["W"], row(p["sc"]["b"])),
        (x2d, None, None, p["mlp1"]["W"], row(p["mlp1"]["b"])),
    ])
    sc_scale, sc_shift = _bn_fold(s_sc, q_sc, N, p["sc"]["g"], p["sc"]["be"])
    h1_scale, h1_shift = _bn_fold(s1, q1, N, p["mlp1"]["g"], p["mlp1"]["be"])

    l1, l2 = p["l1"], p["l2"]
    a_dg1, c_dg1 = _split3(l1["dg"]["W"], 16)
    a_en1, c_en1, wd1 = _split_enc(l1["enc"]["W"])
    sc_arr, ae1, u1, tab1 = _s2b(
        t_sc, t1, pos8,
        [sc_scale, sc_shift, h1_scale, h1_shift,
         a_dg1, c_dg1, row(l1["dg"]["b"]),
         a_en1, c_en1, row(l1["enc"]["b"])])

    gath1 = _sc_gather_rows(tab1, idx3, 128)
    se1, qe1, sd1, qd1 = _edge_stats(gath1, ae1, u1, pos8, wd1, 16, 128)
    enc1_sc, enc1_sh = _bn_fold(se1, qe1, E, l1["enc"]["g"], l1["enc"]["be"])
    dg1_sc, dg1_sh = _bn_fold(sd1, qd1, E, l1["dg"]["g"], l1["dg"]["be"])
    att1, raw1 = _edge_aggr(gath1, ae1, u1, pos8, wd1, enc1_sc, enc1_sh,
                            dg1_sc, dg1_sh, l1["att"]["W"], 16, 128, True)

    (t_p1, s_p1, q_p1), (t_r1, s_r1, q_r1) = _linstat([
        (att1, None, None, l1["post"]["W"], row(l1["post"]["b"])),
        (raw1, None, None, l1["raw"]["W"], row(l1["raw"]["b"])),
    ])
    p1_scale, p1_shift = _bn_fold(s_p1, q_p1, N, l1["post"]["g"], l1["post"]["be"])
    r1_scale, r1_shift = _bn_fold(s_r1, q_r1, N, l1["raw"]["g"], l1["raw"]["be"])

    a_dg2, c_dg2 = _split3(l2["dg"]["W"], 32)
    a_en2, c_en2, wd2 = _split_enc(l2["enc"]["W"])
    rec, ae2, u2, tab2 = _s6b(
        t_p1, t_r1, pos8,
        [p1_scale, p1_shift, r1_scale, r1_shift,
         a_dg2, c_dg2, row(l2["dg"]["b"]),
         a_en2, c_en2, row(l2["enc"]["b"])])

    gath2 = _sc_gather_rows(tab2, idx3, 256)
    se2, qe2, sd2, qd2 = _edge_stats(gath2, ae2, u2, pos8, wd2, 32, 256)
    enc2_sc, enc2_sh = _bn_fold(se2, qe2, E, l2["enc"]["g"], l2["enc"]["be"])
    dg2_sc, dg2_sh = _bn_fold(sd2, qd2, E, l2["dg"]["g"], l2["dg"]["be"])
    att2 = _edge_aggr(gath2, ae2, u2, pos8, wd2, enc2_sc, enc2_sh,
                      dg2_sc, dg2_sh, l2["att"]["W"], 32, 256, False)[0]

    [(t_p2, s_p2, q_p2)] = _linstat([
        (att2, None, None, l2["post"]["W"], row(l2["post"]["b"])),
    ])
    p2_scale, p2_shift = _bn_fold(s_p2, q_p2, N, l2["post"]["g"], l2["post"]["be"])

    [(t_m, s_m, q_m)] = _linstat([
        (t_p2, p2_scale, p2_shift, p["mlp2"]["W"], row(p["mlp2"]["b"])),
    ])
    m_scale, m_shift = _bn_fold(s_m, q_m, N, p["mlp2"]["g"], p["mlp2"]["be"])

    out = _s10c(t_m, sc_arr, m_scale, m_shift)

    return (out.reshape(B, N, 128), pos2d.reshape(B, N, 3),
            rec.reshape(B, N, 128))


# speedup vs baseline: 5.5766x; 1.5929x over previous
import functools

import jax
import jax.numpy as jnp
from jax import lax
from jax.experimental import pallas as pl
from jax.experimental.pallas import tpu as pltpu
from jax.experimental.pallas import tpu_sc as plsc

N = 10000
K = 16
E = N * K
NW = 32
CHUNK = 128
E_PAD = 163840
CH_PER_W = E_PAD // (NW * CHUNK)
T = 400
NT = N // T
RT = 2000
NRT = N // RT
QT = 256
NPAD = 10240
NB = NPAD // 128
DEPTH = 6

f32 = jnp.float32
i32 = jnp.int32


def _lrelu(h):
    return jnp.where(h > 0, h, 0.2 * h)


def _knn_body(pp_ref, qt_ref, out_ref, d3_ref, cd_ref, ci_ref):
    pp = pp_ref[...]
    qt = qt_ref[...]
    pn = jnp.sum(pp * pp, axis=1, keepdims=True)
    qn = jnp.sum(qt * qt, axis=0, keepdims=True)
    mm = jnp.dot(pp, qt, preferred_element_type=f32)
    d3_ref[...] = (pn + qn - 2.0 * mm).reshape(NB, 128, QT)

    rowio = lax.broadcasted_iota(i32, (NB, 128, QT), 1)
    blkio = lax.broadcasted_iota(i32, (NB, 1, QT), 0)

    def level(l, carry):
        d3 = d3_ref[...]
        m = jnp.min(d3, axis=1, keepdims=True)
        r = jnp.min(jnp.where(d3 == m, rowio, i32(128)),
                    axis=1, keepdims=True)
        d3_ref[...] = jnp.where(rowio == r, f32(3e38), d3)
        cd_ref[pl.ds(l * NB, NB), :] = m.reshape(NB, QT)
        ci_ref[pl.ds(l * NB, NB), :] = (r + blkio * 128).reshape(NB, QT)
        return carry

    lax.fori_loop(0, DEPTH, level, 0)

    cio = lax.broadcasted_iota(i32, (DEPTH * NB, QT), 0)
    k16 = lax.broadcasted_iota(i32, (K, QT), 0)

    def merge(k, acc):
        cd = cd_ref[...]
        ci = ci_ref[...]
        m = jnp.min(cd, axis=0, keepdims=True)
        eq = cd == m
        idx = jnp.min(jnp.where(eq, ci, i32(NPAD)), axis=0, keepdims=True)
        p = jnp.min(jnp.where(eq & (ci == idx), cio, i32(DEPTH * NB)),
                    axis=0, keepdims=True)
        cd_ref[...] = jnp.where(cio == p, f32(3e38), cd)
        return jnp.where(k16 == k, idx, acc)

    out_ref[...] = lax.fori_loop(0, K, merge, jnp.zeros((K, QT), i32))


def _knn(pos_pad, q8T):
    return pl.pallas_call(
        _knn_body,
        grid=(NPAD // QT,),
        in_specs=[
            pl.BlockSpec((NPAD, 8), lambda i: (0, 0)),
            pl.BlockSpec((8, QT), lambda i: (0, i)),
        ],
        out_specs=pl.BlockSpec((K, QT), lambda i: (0, i)),
        out_shape=jax.ShapeDtypeStruct((K, NPAD), i32),
        scratch_shapes=[pltpu.VMEM((NB, 128, QT), f32),
                        pltpu.VMEM((DEPTH * NB, QT), f32),
                        pltpu.VMEM((DEPTH * NB, QT), i32)],
    )(pos_pad, q8T)


def _linstat(heads):
    in_specs, args, douts = [], [], []
    for (x, sc, sh, w, b) in heads:
        din, dout = w.shape
        douts.append(dout)
        in_specs.append(pl.BlockSpec((RT, din), lambda i: (i, 0)))
        args.append(x)
        if sc is not None:
            in_specs.append(pl.BlockSpec((1, din), lambda i: (0, 0)))
            args.append(sc)
            in_specs.append(pl.BlockSpec((1, din), lambda i: (0, 0)))
            args.append(sh)
        in_specs.append(pl.BlockSpec((din, dout), lambda i: (0, 0)))
        args.append(w)
        in_specs.append(pl.BlockSpec((1, dout), lambda i: (0, 0)))
        args.append(b)

    out_specs, out_shapes = [], []
    for dout in douts:
        out_specs += [
            pl.BlockSpec((RT, dout), lambda i: (i, 0)),
            pl.BlockSpec((1, dout), lambda i: (0, 0)),
            pl.BlockSpec((1, dout), lambda i: (0, 0)),
        ]
        out_shapes += [
            jax.ShapeDtypeStruct((N, dout), f32),
            jax.ShapeDtypeStruct((1, dout), f32),
            jax.ShapeDtypeStruct((1, dout), f32),
        ]

    has_pre = [h[1] is not None for h in heads]

    def body(*refs):
        i = pl.program_id(0)
        pos = 0
        ins = []
        for hp in has_pre:
            n_in = 5 if hp else 3
            ins.append(refs[pos:pos + n_in])
            pos += n_in
        outs = refs[pos:]
        for hi, hrefs in enumerate(ins):
            if has_pre[hi]:
                x_ref, sc_ref, sh_ref, w_ref, b_ref = hrefs
                x = _lrelu(x_ref[...] * sc_ref[...] + sh_ref[...])
            else:
                x_ref, w_ref, b_ref = hrefs
                x = x_ref[...]
            t = jnp.dot(x, w_ref[...], preferred_element_type=f32) + b_ref[...]
            t_ref, s_ref, q_ref = outs[3 * hi:3 * hi + 3]
            t_ref[...] = t

            @pl.when(i == 0)
            def _():
                s_ref[...] = jnp.zeros_like(s_ref)
                q_ref[...] = jnp.zeros_like(q_ref)

            s_ref[...] += jnp.sum(t, axis=0, keepdims=True)
            q_ref[...] += jnp.sum(t * t, axis=0, keepdims=True)

    flat = pl.pallas_call(
        body,
        grid=(NRT,),
        in_specs=in_specs,
        out_specs=out_specs,
        out_shape=out_shapes,
    )(*args)
    return [tuple(flat[3 * i:3 * i + 3]) for i in range(len(heads))]


def _bn_fold(s, q, n, g, be):
    m = s / n
    v = q / n - m * m
    scale = g[None, :] / jnp.sqrt(v + 1e-6)
    shift = be[None, :] - m * scale
    return scale, shift


def _s2b_body(tsc_ref, t1_ref, p8_ref, scs_ref, sch_ref, h1s_ref, h1h_ref,
              adg_ref, cdg_ref, bdg_ref, aen_ref, cen_ref, ben_ref,
              sc_out, ae1_out, u1_out, tab_out):
    sc_out[...] = tsc_ref[...] * scs_ref[...] + sch_ref[...]
    h1 = _lrelu(t1_ref[...] * h1s_ref[...] + h1h_ref[...])
    p8 = p8_ref[...]
    ae1_out[...] = jnp.dot(p8, aen_ref[...], preferred_element_type=f32) + ben_ref[...]
    u1_out[...] = jnp.dot(h1, adg_ref[...], preferred_element_type=f32) + bdg_ref[...]
    ce1 = jnp.dot(p8, cen_ref[...], preferred_element_type=f32)
    w1 = jnp.dot(h1, cdg_ref[...], preferred_element_type=f32)
    tab_out[...] = jnp.concatenate(
        [h1, ce1, w1, p8[:, 0:3], jnp.zeros((RT, 61), f32)], axis=1)


def _s2b(t_sc, t1, pos8, consts):
    vec = lambda c: pl.BlockSpec((1, c), lambda i: (0, 0))
    mat = lambda a, b: pl.BlockSpec((a, b), lambda i: (0, 0))
    return pl.pallas_call(
        _s2b_body,
        grid=(NRT,),
        in_specs=[
            pl.BlockSpec((RT, 128), lambda i: (i, 0)),
            pl.BlockSpec((RT, 16), lambda i: (i, 0)),
            pl.BlockSpec((RT, 8), lambda i: (i, 0)),
            vec(128), vec(128), vec(16), vec(16),
            mat(16, 32), mat(16, 32), vec(32),
            mat(8, 16), mat(8, 16), vec(16),
        ],
        out_specs=[
            pl.BlockSpec((RT, 128), lambda i: (i, 0)),
            pl.BlockSpec((RT, 16), lambda i: (i, 0)),
            pl.BlockSpec((RT, 32), lambda i: (i, 0)),
            pl.BlockSpec((RT, 128), lambda i: (i, 0)),
        ],
        out_shape=[
            jax.ShapeDtypeStruct((N, 128), f32),
            jax.ShapeDtypeStruct((N, 16), f32),
            jax.ShapeDtypeStruct((N, 32), f32),
            jax.ShapeDtypeStruct((N, 128), f32),
        ],
    )(t_sc, t1, pos8, *consts)


def _sc_gather_rows(table, idx3, rowlen):
    mesh = plsc.VectorSubcoreMesh(core_axis_name="c", subcore_axis_name="s")

    def body(idx_hbm, tab_hbm, out_hbm, idx0, idx1, rows0, rows1, sem0, sem1):
        wid = lax.axis_index("s") * 2 + lax.axis_index("c")
        base = wid * CH_PER_W

        def start(iv, rv, sem, c):
            pltpu.sync_copy(idx_hbm.at[wid, c], iv)
            pltpu.async_copy(tab_hbm.at[iv], rv, sem)

        def finish(iv, rv, sem, c):
            pltpu.make_async_copy(tab_hbm.at[iv], rv, sem).wait()
            pltpu.sync_copy(rv, out_hbm.at[pl.ds((base + c) * CHUNK, CHUNK)])

        start(idx0, rows0, sem0, 0)

        def pair(i, carry):
            c = 2 * i
            start(idx1, rows1, sem1, c + 1)
            finish(idx0, rows0, sem0, c)

            @pl.when(c + 2 < CH_PER_W)
            def _():
                start(idx0, rows0, sem0, c + 2)

            finish(idx1, rows1, sem1, c + 1)
            return carry

        lax.fori_loop(0, CH_PER_W // 2, pair, 0)

    return pl.kernel(
        body,
        out_type=jax.ShapeDtypeStruct((E_PAD, rowlen), f32),
        mesh=mesh,
        scratch_types=[
            pltpu.VMEM((CHUNK,), i32),
            pltpu.VMEM((CHUNK,), i32),
            pltpu.VMEM((CHUNK, rowlen), f32),
            pltpu.VMEM((CHUNK, rowlen), f32),
            pltpu.SemaphoreType.DMA,
            pltpu.SemaphoreType.DMA,
        ],
    )(idx3, table)


def _edge_pre(g_ref, ae_ref, u_ref, p8_ref, wd_ref, c, rowlen):
    g3 = g_ref[...].reshape(T, K, rowlen)
    xj3 = g3[:, :, 0:c]
    cj3 = g3[:, :, c:2 * c]
    wj3 = g3[:, :, 2 * c:4 * c]
    pj3 = g3[:, :, 4 * c:4 * c + 3]
    pi3 = p8_ref[...][:, None, 0:3]
    pd3 = pj3 - pi3
    dist3 = jnp.sqrt(jnp.maximum(
        jnp.sum(pd3 * pd3, axis=2, keepdims=True), 1e-12))
    wd3 = wd_ref[...].reshape(1, 1, c)
    henc3 = ae_ref[...][:, None, :] + cj3 + dist3 * wd3
    hdg3 = u_ref[...][:, None, :] + wj3
    return xj3, henc3, hdg3


def _edge_stats(gath, ae, u, pos8, wd, c, rowlen):
    def body(g_ref, ae_ref, u_ref, p8_ref, wd_ref, se, qe, sd, qd):
        i = pl.program_id(0)
        _, henc3, hdg3 = _edge_pre(g_ref, ae_ref, u_ref, p8_ref, wd_ref, c, rowlen)

        @pl.when(i == 0)
        def _():
            se[...] = jnp.zeros_like(se)
            qe[...] = jnp.zeros_like(qe)
            sd[...] = jnp.zeros_like(sd)
            qd[...] = jnp.zeros_like(qd)

        se[...] += jnp.sum(jnp.sum(henc3, axis=1), axis=0, keepdims=True)
        qe[...] += jnp.sum(jnp.sum(henc3 * henc3, axis=1), axis=0, keepdims=True)
        sd[...] += jnp.sum(jnp.sum(hdg3, axis=1), axis=0, keepdims=True)
        qd[...] += jnp.sum(jnp.sum(hdg3 * hdg3, axis=1), axis=0, keepdims=True)

    return pl.pallas_call(
        body,
        grid=(NT,),
        in_specs=[
            pl.BlockSpec((K * T, rowlen), lambda i: (i, 0)),
            pl.BlockSpec((T, c), lambda i: (i, 0)),
            pl.BlockSpec((T, 2 * c), lambda i: (i, 0)),
            pl.BlockSpec((T, 8), lambda i: (i, 0)),
            pl.BlockSpec((1, c), lambda i: (0, 0)),
        ],
        out_specs=[pl.BlockSpec((1, c), lambda i: (0, 0)),
                   pl.BlockSpec((1, c), lambda i: (0, 0)),
                   pl.BlockSpec((1, 2 * c), lambda i: (0, 0)),
                   pl.BlockSpec((1, 2 * c), lambda i: (0, 0))],
        out_shape=[jax.ShapeDtypeStruct((1, c), f32),
                   jax.ShapeDtypeStruct((1, c), f32),
                   jax.ShapeDtypeStruct((1, 2 * c), f32),
                   jax.ShapeDtypeStruct((1, 2 * c), f32)],
    )(gath, ae, u, pos8, wd)


def _edge_aggr(gath, ae, u, pos8, wd, enc_sc, enc_sh, dg_sc, dg_sh, watt,
               c, rowlen, with_raw):
    def body(g_ref, ae_ref, u_ref, p8_ref, wd_ref, es_ref, eh_ref, ds_ref,
             dh_ref, w_ref, att_out, *maybe_raw):
        xj3, henc3, hdg3 = _edge_pre(g_ref, ae_ref, u_ref, p8_ref, wd_ref, c, rowlen)
        lse3 = _lrelu(henc3 * es_ref[...].reshape(1, 1, c)
                      + eh_ref[...].reshape(1, 1, c))
        dg3 = _lrelu(hdg3 * ds_ref[...].reshape(1, 1, 2 * c)
                     + dh_ref[...].reshape(1, 1, 2 * c))
        local3 = jnp.concatenate([dg3, xj3, lse3], axis=2)
        att2 = jnp.dot(local3.reshape(T * K, 4 * c), w_ref[...],
                       preferred_element_type=f32)
        att3 = att2.reshape(T, K, 4 * c)
        m3 = jnp.max(att3, axis=1, keepdims=True)
        e3 = jnp.exp(att3 - m3)
        ssum3 = jnp.sum(e3, axis=1, keepdims=True)
        scores3 = e3 / (ssum3 + 1e-16)
        att_out[...] = jnp.sum(scores3 * local3, axis=1)
        if with_raw:
            maybe_raw[0][...] = jnp.sum(local3, axis=1)

    out_specs = [pl.BlockSpec((T, 4 * c), lambda i: (i, 0))]
    out_shape = [jax.ShapeDtypeStruct((N, 4 * c), f32)]
    if with_raw:
        out_specs.append(pl.BlockSpec((T, 4 * c), lambda i: (i, 0)))
        out_shape.append(jax.ShapeDtypeStruct((N, 4 * c), f32))

    return pl.pallas_call(
        body,
        grid=(NT,),
        in_specs=[
            pl.BlockSpec((K * T, rowlen), lambda i: (i, 0)),
            pl.BlockSpec((T, c), lambda i: (i, 0)),
            pl.BlockSpec((T, 2 * c), lambda i: (i, 0)),
            pl.BlockSpec((T, 8), lambda i: (i, 0)),
            pl.BlockSpec((1, c), lambda i: (0, 0)),
            pl.BlockSpec((1, c), lambda i: (0, 0)),
            pl.BlockSpec((1, c), lambda i: (0, 0)),
            pl.BlockSpec((1, 2 * c), lambda i: (0, 0)),
            pl.BlockSpec((1, 2 * c), lambda i: (0, 0)),
            pl.BlockSpec((4 * c, 4 * c), lambda i: (0, 0)),
        ],
        out_specs=out_specs,
        out_shape=out_shape,
    )(gath, ae, u, pos8, wd, enc_sc, enc_sh, dg_sc, dg_sh, watt)


def _s6b_body(tp_ref, tr_ref, p8_ref, ps_ref, ph_ref, rs_ref, rh_ref,
              adg_ref, cdg_ref, bdg_ref, aen_ref, cen_ref, ben_ref,
              rec_out, ae2_out, u2_out, tab_out):
    h2 = _lrelu(tp_ref[...] * ps_ref[...] + ph_ref[...])
    rec_out[...] = _lrelu(tr_ref[...] * rs_ref[...] + rh_ref[...])
    p8 = p8_ref[...]
    ae2_out[...] = jnp.dot(p8, aen_ref[...], preferred_element_type=f32) + ben_ref[...]
    u2_out[...] = jnp.dot(h2, adg_ref[...], preferred_element_type=f32) + bdg_ref[...]
    ce2 = jnp.dot(p8, cen_ref[...], preferred_element_type=f32)
    w2 = jnp.dot(h2, cdg_ref[...], preferred_element_type=f32)
    tab_out[...] = jnp.concatenate(
        [h2, ce2, w2, p8[:, 0:3], jnp.zeros((RT, 125), f32)], axis=1)


def _s6b(t_post, t_raw, pos8, consts):
    vec = lambda c: pl.BlockSpec((1, c), lambda i: (0, 0))
    mat = lambda a, b: pl.BlockSpec((a, b), lambda i: (0, 0))
    return pl.pallas_call(
        _s6b_body,
        grid=(NRT,),
        in_specs=[
            pl.BlockSpec((RT, 32), lambda i: (i, 0)),
            pl.BlockSpec((RT, 128), lambda i: (i, 0)),
            pl.BlockSpec((RT, 8), lambda i: (i, 0)),
            vec(32), vec(32), vec(128), vec(128),
            mat(32, 64), mat(32, 64), vec(64),
            mat(8, 32), mat(8, 32), vec(32),
        ],
        out_specs=[
            pl.BlockSpec((RT, 128), lambda i: (i, 0)),
            pl.BlockSpec((RT, 32), lambda i: (i, 0)),
            pl.BlockSpec((RT, 64), lambda i: (i, 0)),
            pl.BlockSpec((RT, 256), lambda i: (i, 0)),
        ],
        out_shape=[
            jax.ShapeDtypeStruct((N, 128), f32),
            jax.ShapeDtypeStruct((N, 32), f32),
            jax.ShapeDtypeStruct((N, 64), f32),
            jax.ShapeDtypeStruct((N, 256), f32),
        ],
    )(t_post, t_raw, pos8, *consts)


def _s10c_body(tm_ref, sc_ref, s_ref, h_ref, out_ref):
    out_ref[...] = _lrelu(tm_ref[...] * s_ref[...] + h_ref[...] + sc_ref[...])


def _s10c(t_m, sc, scale, shift):
    return pl.pallas_call(
        _s10c_body,
        grid=(NRT,),
        in_specs=[
            pl.BlockSpec((RT, 128), lambda i: (i, 0)),
            pl.BlockSpec((RT, 128), lambda i: (i, 0)),
            pl.BlockSpec((1, 128), lambda i: (0, 0)),
            pl.BlockSpec((1, 128), lambda i: (0, 0)),
        ],
        out_specs=pl.BlockSpec((RT, 128), lambda i: (i, 0)),
        out_shape=jax.ShapeDtypeStruct((N, 128), f32),
    )(t_m, sc, scale, shift)


def _split3(w, c):
    a = w[0:c] - w[2 * c:3 * c]
    cc = w[c:2 * c] + w[2 * c:3 * c]
    return a, cc


def _split_enc(w):
    a = w[0:3] - w[6:9]
    cc = w[3:6] + w[6:9]
    pad = lambda m: jnp.concatenate([m, jnp.zeros((5, m.shape[1]), f32)], axis=0)
    return pad(a), pad(cc), w[9:10]


def kernel(pos, x, params):
    B = pos.shape[0]
    p = params
    pos2d = pos.reshape(N, 3)
    x2d = x.reshape(N, 128)
    pos8 = jnp.concatenate([pos2d, jnp.zeros((N, 5), f32)], axis=1)
    pos_pad = jnp.concatenate(
        [pos8, jnp.full((NPAD - N, 8), 1e3, f32)], axis=0)
    q8T = jnp.concatenate(
        [pos2d.T, jnp.full((3, NPAD - N), 1e3, f32)], axis=1)
    q8T = jnp.concatenate([q8T, jnp.zeros((5, NPAD), f32)], axis=0)

    nbrT = _knn(pos_pad, q8T)
    idx_flat = nbrT[:, :N].T.reshape(E)
    idx3 = jnp.concatenate(
        [idx_flat, jnp.zeros((E_PAD - E,), i32)]).reshape(NW, CH_PER_W, CHUNK)

    row = lambda v: v[None, :]

    (t_sc, s_sc, q_sc), (t1, s1, q1) = _linstat([
        (x2d, None, None, p["sc"]["W"], row(p["sc"]["b"])),
        (x2d, None, None, p["mlp1"]["W"], row(p["mlp1"]["b"])),
    ])
    sc_scale, sc_shift = _bn_fold(s_sc, q_sc, N, p["sc"]["g"], p["sc"]["be"])
    h1_scale, h1_shift = _bn_fold(s1, q1, N, p["mlp1"]["g"], p["mlp1"]["be"])

    l1, l2 = p["l1"], p["l2"]
    a_dg1, c_dg1 = _split3(l1["dg"]["W"], 16)
    a_en1, c_en1, wd1 = _split_enc(l1["enc"]["W"])
    sc_arr, ae1, u1, tab1 = _s2b(
        t_sc, t1, pos8,
        [sc_scale, sc_shift, h1_scale, h1_shift,
         a_dg1, c_dg1, row(l1["dg"]["b"]),
         a_en1, c_en1, row(l1["enc"]["b"])])

    gath1 = _sc_gather_rows(tab1, idx3, 128)
    se1, qe1, sd1, qd1 = _edge_stats(gath1, ae1, u1, pos8, wd1, 16, 128)
    enc1_sc, enc1_sh = _bn_fold(se1, qe1, E, l1["enc"]["g"], l1["enc"]["be"])
    dg1_sc, dg1_sh = _bn_fold(sd1, qd1, E, l1["dg"]["g"], l1["dg"]["be"])
    att1, raw1 = _edge_aggr(gath1, ae1, u1, pos8, wd1, enc1_sc, enc1_sh,
                            dg1_sc, dg1_sh, l1["att"]["W"], 16, 128, True)

    (t_p1, s_p1, q_p1), (t_r1, s_r1, q_r1) = _linstat([
        (att1, None, None, l1["post"]["W"], row(l1["post"]["b"])),
        (raw1, None, None, l1["raw"]["W"], row(l1["raw"]["b"])),
    ])
    p1_scale, p1_shift = _bn_fold(s_p1, q_p1, N, l1["post"]["g"], l1["post"]["be"])
    r1_scale, r1_shift = _bn_fold(s_r1, q_r1, N, l1["raw"]["g"], l1["raw"]["be"])

    a_dg2, c_dg2 = _split3(l2["dg"]["W"], 32)
    a_en2, c_en2, wd2 = _split_enc(l2["enc"]["W"])
    rec, ae2, u2, tab2 = _s6b(
        t_p1, t_r1, pos8,
        [p1_scale, p1_shift, r1_scale, r1_shift,
         a_dg2, c_dg2, row(l2["dg"]["b"]),
         a_en2, c_en2, row(l2["enc"]["b"])])

    gath2 = _sc_gather_rows(tab2, idx3, 256)
    se2, qe2, sd2, qd2 = _edge_stats(gath2, ae2, u2, pos8, wd2, 32, 256)
    enc2_sc, enc2_sh = _bn_fold(se2, qe2, E, l2["enc"]["g"], l2["enc"]["be"])
    dg2_sc, dg2_sh = _bn_fold(sd2, qd2, E, l2["dg"]["g"], l2["dg"]["be"])
    att2 = _edge_aggr(gath2, ae2, u2, pos8, wd2, enc2_sc, enc2_sh,
                      dg2_sc, dg2_sh, l2["att"]["W"], 32, 256, False)[0]

    [(t_p2, s_p2, q_p2)] = _linstat([
        (att2, None, None, l2["post"]["W"], row(l2["post"]["b"])),
    ])
    p2_scale, p2_shift = _bn_fold(s_p2, q_p2, N, l2["post"]["g"], l2["post"]["be"])

    [(t_m, s_m, q_m)] = _linstat([
        (t_p2, p2_scale, p2_shift, p["mlp2"]["W"], row(p["mlp2"]["b"])),
    ])
    m_scale, m_shift = _bn_fold(s_m, q_m, N, p["mlp2"]["g"], p["mlp2"]["be"])

    out = _s10c(t_m, sc_arr, m_scale, m_shift)

    return (out.reshape(B, N, 128), pos2d.reshape(B, N, 3),
            rec.reshape(B, N, 128))


# speedup vs baseline: 6.2990x; 1.1295x over previous
import functools

import jax
import jax.numpy as jnp
from jax import lax
from jax.experimental import pallas as pl
from jax.experimental.pallas import tpu as pltpu
from jax.experimental.pallas import tpu_sc as plsc

N = 10000
K = 16
E = N * K
NW = 32
CHUNK = 128
E_PAD = 163840
CH_PER_W = E_PAD // (NW * CHUNK)
T = 400
NT = N // T
RT = 2000
NRT = N // RT
QT = 256
NPAD = 10240
NB = NPAD // 128
DEPTH = 6

f32 = jnp.float32
i32 = jnp.int32


def _lrelu(h):
    return jnp.where(h > 0, h, 0.2 * h)


def _knn_body(pp_ref, qt_ref, out_ref, d3_ref, cd_ref, ci_ref):
    pp = pp_ref[...]
    qt = qt_ref[...]
    pn = jnp.sum(pp * pp, axis=1, keepdims=True)
    qn = jnp.sum(qt * qt, axis=0, keepdims=True)
    mm = jnp.dot(pp, qt, preferred_element_type=f32)
    d3_ref[...] = (pn + qn - 2.0 * mm).reshape(NB, 128, QT)

    rowio = lax.broadcasted_iota(i32, (NB, 128, QT), 1)
    blkio = lax.broadcasted_iota(i32, (NB, 1, QT), 0)

    def level(l, carry):
        d3 = d3_ref[...]
        m = jnp.min(d3, axis=1, keepdims=True)
        r = jnp.min(jnp.where(d3 == m, rowio, i32(128)),
                    axis=1, keepdims=True)
        d3_ref[...] = jnp.where(rowio == r, f32(3e38), d3)
        cd_ref[pl.ds(l * NB, NB), :] = m.reshape(NB, QT)
        ci_ref[pl.ds(l * NB, NB), :] = (r + blkio * 128).reshape(NB, QT)
        return carry

    lax.fori_loop(0, DEPTH, level, 0)

    cio = lax.broadcasted_iota(i32, (DEPTH * NB, QT), 0)
    k16 = lax.broadcasted_iota(i32, (K, QT), 0)

    def merge(k, acc):
        cd = cd_ref[...]
        ci = ci_ref[...]
        m = jnp.min(cd, axis=0, keepdims=True)
        eq = cd == m
        idx = jnp.min(jnp.where(eq, ci, i32(NPAD)), axis=0, keepdims=True)
        p = jnp.min(jnp.where(eq & (ci == idx), cio, i32(DEPTH * NB)),
                    axis=0, keepdims=True)
        cd_ref[...] = jnp.where(cio == p, f32(3e38), cd)
        return jnp.where(k16 == k, idx, acc)

    out_ref[...] = lax.fori_loop(0, K, merge, jnp.zeros((K, QT), i32))


def _knn(pos_pad, q8T):
    return pl.pallas_call(
        _knn_body,
        grid=(NPAD // QT,),
        in_specs=[
            pl.BlockSpec((NPAD, 8), lambda i: (0, 0)),
            pl.BlockSpec((8, QT), lambda i: (0, i)),
        ],
        out_specs=pl.BlockSpec((K, QT), lambda i: (0, i)),
        out_shape=jax.ShapeDtypeStruct((K, NPAD), i32),
        scratch_shapes=[pltpu.VMEM((NB, 128, QT), f32),
                        pltpu.VMEM((DEPTH * NB, QT), f32),
                        pltpu.VMEM((DEPTH * NB, QT), i32)],
    )(pos_pad, q8T)


def _linstat(heads):
    in_specs, args, douts = [], [], []
    for (x, sc, sh, w, b) in heads:
        din, dout = w.shape
        douts.append(dout)
        in_specs.append(pl.BlockSpec((RT, din), lambda i: (i, 0)))
        args.append(x)
        if sc is not None:
            in_specs.append(pl.BlockSpec((1, din), lambda i: (0, 0)))
            args.append(sc)
            in_specs.append(pl.BlockSpec((1, din), lambda i: (0, 0)))
            args.append(sh)
        in_specs.append(pl.BlockSpec((din, dout), lambda i: (0, 0)))
        args.append(w)
        in_specs.append(pl.BlockSpec((1, dout), lambda i: (0, 0)))
        args.append(b)

    out_specs, out_shapes = [], []
    for dout in douts:
        out_specs += [
            pl.BlockSpec((RT, dout), lambda i: (i, 0)),
            pl.BlockSpec((1, dout), lambda i: (0, 0)),
            pl.BlockSpec((1, dout), lambda i: (0, 0)),
        ]
        out_shapes += [
            jax.ShapeDtypeStruct((N, dout), f32),
            jax.ShapeDtypeStruct((1, dout), f32),
            jax.ShapeDtypeStruct((1, dout), f32),
        ]

    has_pre = [h[1] is not None for h in heads]

    def body(*refs):
        i = pl.program_id(0)
        pos = 0
        ins = []
        for hp in has_pre:
            n_in = 5 if hp else 3
            ins.append(refs[pos:pos + n_in])
            pos += n_in
        outs = refs[pos:]
        for hi, hrefs in enumerate(ins):
            if has_pre[hi]:
                x_ref, sc_ref, sh_ref, w_ref, b_ref = hrefs
                x = _lrelu(x_ref[...] * sc_ref[...] + sh_ref[...])
            else:
                x_ref, w_ref, b_ref = hrefs
                x = x_ref[...]
            t = jnp.dot(x, w_ref[...], preferred_element_type=f32) + b_ref[...]
            t_ref, s_ref, q_ref = outs[3 * hi:3 * hi + 3]
            t_ref[...] = t

            @pl.when(i == 0)
            def _():
                s_ref[...] = jnp.zeros_like(s_ref)
                q_ref[...] = jnp.zeros_like(q_ref)

            s_ref[...] += jnp.sum(t, axis=0, keepdims=True)
            q_ref[...] += jnp.sum(t * t, axis=0, keepdims=True)

    flat = pl.pallas_call(
        body,
        grid=(NRT,),
        in_specs=in_specs,
        out_specs=out_specs,
        out_shape=out_shapes,
    )(*args)
    return [tuple(flat[3 * i:3 * i + 3]) for i in range(len(heads))]


def _bn_fold(s, q, n, g, be):
    m = s / n
    v = q / n - m * m
    scale = g[None, :] / jnp.sqrt(v + 1e-6)
    shift = be[None, :] - m * scale
    return scale, shift


def _s2b_body(tsc_ref, t1_ref, p8_ref, scs_ref, sch_ref, h1s_ref, h1h_ref,
              adg_ref, cdg_ref, bdg_ref, aen_ref, cen_ref, ben_ref,
              sc_out, ae1_out, u1_out, tab_out):
    sc_out[...] = tsc_ref[...] * scs_ref[...] + sch_ref[...]
    h1 = _lrelu(t1_ref[...] * h1s_ref[...] + h1h_ref[...])
    p8 = p8_ref[...]
    ae1_out[...] = jnp.dot(p8, aen_ref[...], preferred_element_type=f32) + ben_ref[...]
    u1_out[...] = jnp.dot(h1, adg_ref[...], preferred_element_type=f32) + bdg_ref[...]
    ce1 = jnp.dot(p8, cen_ref[...], preferred_element_type=f32)
    w1 = jnp.dot(h1, cdg_ref[...], preferred_element_type=f32)
    tab_out[...] = jnp.concatenate(
        [h1, ce1, w1, p8[:, 0:3], jnp.zeros((RT, 61), f32)], axis=1)


def _s2b(t_sc, t1, pos8, consts):
    vec = lambda c: pl.BlockSpec((1, c), lambda i: (0, 0))
    mat = lambda a, b: pl.BlockSpec((a, b), lambda i: (0, 0))
    return pl.pallas_call(
        _s2b_body,
        grid=(NRT,),
        in_specs=[
            pl.BlockSpec((RT, 128), lambda i: (i, 0)),
            pl.BlockSpec((RT, 16), lambda i: (i, 0)),
            pl.BlockSpec((RT, 8), lambda i: (i, 0)),
            vec(128), vec(128), vec(16), vec(16),
            mat(16, 32), mat(16, 32), vec(32),
            mat(8, 16), mat(8, 16), vec(16),
        ],
        out_specs=[
            pl.BlockSpec((RT, 128), lambda i: (i, 0)),
            pl.BlockSpec((RT, 16), lambda i: (i, 0)),
            pl.BlockSpec((RT, 32), lambda i: (i, 0)),
            pl.BlockSpec((RT, 128), lambda i: (i, 0)),
        ],
        out_shape=[
            jax.ShapeDtypeStruct((N, 128), f32),
            jax.ShapeDtypeStruct((N, 16), f32),
            jax.ShapeDtypeStruct((N, 32), f32),
            jax.ShapeDtypeStruct((N, 128), f32),
        ],
    )(t_sc, t1, pos8, *consts)


def _sc_gather_rows(table, idx3, rowlen):
    mesh = plsc.VectorSubcoreMesh(core_axis_name="c", subcore_axis_name="s")

    def body(idx_hbm, tab_hbm, out_hbm, idx0, idx1, rows0, rows1, sem0, sem1):
        wid = lax.axis_index("s") * 2 + lax.axis_index("c")
        base = wid * CH_PER_W

        def start(iv, rv, sem, c):
            pltpu.sync_copy(idx_hbm.at[wid, c], iv)
            pltpu.async_copy(tab_hbm.at[iv], rv, sem)

        def finish(iv, rv, sem, c):
            pltpu.make_async_copy(tab_hbm.at[iv], rv, sem).wait()
            pltpu.sync_copy(rv, out_hbm.at[pl.ds((base + c) * CHUNK, CHUNK)])

        start(idx0, rows0, sem0, 0)

        def pair(i, carry):
            c = 2 * i
            start(idx1, rows1, sem1, c + 1)
            finish(idx0, rows0, sem0, c)

            @pl.when(c + 2 < CH_PER_W)
            def _():
                start(idx0, rows0, sem0, c + 2)

            finish(idx1, rows1, sem1, c + 1)
            return carry

        lax.fori_loop(0, CH_PER_W // 2, pair, 0)

    return pl.kernel(
        body,
        out_type=jax.ShapeDtypeStruct((E_PAD, rowlen), f32),
        mesh=mesh,
        scratch_types=[
            pltpu.VMEM((CHUNK,), i32),
            pltpu.VMEM((CHUNK,), i32),
            pltpu.VMEM((CHUNK, rowlen), f32),
            pltpu.VMEM((CHUNK, rowlen), f32),
            pltpu.SemaphoreType.DMA,
            pltpu.SemaphoreType.DMA,
        ],
    )(idx3, table)


def _edge_pre(g3, ae_ref, u_ref, wd_ref, c, dist3):
    henc3 = (ae_ref[...][:, None, :] + g3[:, :, c:2 * c]
             + dist3 * wd_ref[...].reshape(1, 1, c))
    hdg3 = u_ref[...][:, None, :] + g3[:, :, 2 * c:4 * c]
    return henc3, hdg3


def _edge_stats(gath, ae, u, dist_or_pos, wd, c, rl, first):
    def body(g_ref, ae_ref, u_ref, dp_ref, wd_ref, se, qe, sd, qd, *dout):
        i = pl.program_id(0)
        g3 = g_ref[...].reshape(T, K, rl)
        if first:
            pd3 = g3[:, :, 4 * c:4 * c + 3] - dp_ref[...][:, None, 0:3]
            dist3 = jnp.sqrt(jnp.maximum(
                jnp.sum(pd3 * pd3, axis=2, keepdims=True), 1e-12))
            dout[0][...] = jnp.broadcast_to(dist3, (T, K, 8))
        else:
            dist3 = dp_ref[...][:, :, 0:1]
        henc3, hdg3 = _edge_pre(g3, ae_ref, u_ref, wd_ref, c, dist3)

        @pl.when(i == 0)
        def _():
            se[...] = jnp.zeros_like(se)
            qe[...] = jnp.zeros_like(qe)
            sd[...] = jnp.zeros_like(sd)
            qd[...] = jnp.zeros_like(qd)

        se[...] += jnp.sum(jnp.sum(henc3, axis=1), axis=0, keepdims=True)
        qe[...] += jnp.sum(jnp.sum(henc3 * henc3, axis=1), axis=0, keepdims=True)
        sd[...] += jnp.sum(jnp.sum(hdg3, axis=1), axis=0, keepdims=True)
        qd[...] += jnp.sum(jnp.sum(hdg3 * hdg3, axis=1), axis=0, keepdims=True)

    dp_spec = (pl.BlockSpec((T, 8), lambda i: (i, 0)) if first
               else pl.BlockSpec((T, K, 8), lambda i: (i, 0, 0)))
    out_specs = [pl.BlockSpec((1, c), lambda i: (0, 0)),
                 pl.BlockSpec((1, c), lambda i: (0, 0)),
                 pl.BlockSpec((1, 2 * c), lambda i: (0, 0)),
                 pl.BlockSpec((1, 2 * c), lambda i: (0, 0))]
    out_shape = [jax.ShapeDtypeStruct((1, c), f32),
                 jax.ShapeDtypeStruct((1, c), f32),
                 jax.ShapeDtypeStruct((1, 2 * c), f32),
                 jax.ShapeDtypeStruct((1, 2 * c), f32)]
    if first:
        out_specs.append(pl.BlockSpec((T, K, 8), lambda i: (i, 0, 0)))
        out_shape.append(jax.ShapeDtypeStruct((N, K, 8), f32))

    return pl.pallas_call(
        body,
        grid=(NT,),
        in_specs=[
            pl.BlockSpec((K * T, rl), lambda i: (i, 0)),
            pl.BlockSpec((T, c), lambda i: (i, 0)),
            pl.BlockSpec((T, 2 * c), lambda i: (i, 0)),
            dp_spec,
            pl.BlockSpec((1, c), lambda i: (0, 0)),
        ],
        out_specs=out_specs,
        out_shape=out_shape,
    )(gath, ae, u, dist_or_pos, wd)


def _edge_aggr(gath, ae, u, dist3d, wd, enc_sc, enc_sh, dg_sc, dg_sh, watt,
               c, rl, with_raw):
    def body(g_ref, ae_ref, u_ref, d_ref, wd_ref, es_ref, eh_ref, ds_ref,
             dh_ref, w_ref, att_out, *maybe_raw):
        g3 = g_ref[...].reshape(T, K, rl)
        xj3 = g3[:, :, 0:c]
        dist3 = d_ref[...][:, :, 0:1]
        henc3, hdg3 = _edge_pre(g3, ae_ref, u_ref, wd_ref, c, dist3)
        lse3 = _lrelu(henc3 * es_ref[...].reshape(1, 1, c)
                      + eh_ref[...].reshape(1, 1, c))
        dg3 = _lrelu(hdg3 * ds_ref[...].reshape(1, 1, 2 * c)
                     + dh_ref[...].reshape(1, 1, 2 * c))
        local3 = jnp.concatenate([dg3, xj3, lse3], axis=2)
        att2 = jnp.dot(local3.reshape(T * K, 4 * c), w_ref[...],
                       preferred_element_type=f32)
        att3 = att2.reshape(T, K, 4 * c)
        m3 = jnp.max(att3, axis=1, keepdims=True)
        e3 = jnp.exp(att3 - m3)
        ssum3 = jnp.sum(e3, axis=1, keepdims=True)
        scores3 = e3 / (ssum3 + 1e-16)
        att_out[...] = jnp.sum(scores3 * local3, axis=1)
        if with_raw:
            maybe_raw[0][...] = jnp.sum(local3, axis=1)

    out_specs = [pl.BlockSpec((T, 4 * c), lambda i: (i, 0))]
    out_shape = [jax.ShapeDtypeStruct((N, 4 * c), f32)]
    if with_raw:
        out_specs.append(pl.BlockSpec((T, 4 * c), lambda i: (i, 0)))
        out_shape.append(jax.ShapeDtypeStruct((N, 4 * c), f32))

    return pl.pallas_call(
        body,
        grid=(NT,),
        in_specs=[
            pl.BlockSpec((K * T, rl), lambda i: (i, 0)),
            pl.BlockSpec((T, c), lambda i: (i, 0)),
            pl.BlockSpec((T, 2 * c), lambda i: (i, 0)),
            pl.BlockSpec((T, K, 8), lambda i: (i, 0, 0)),
            pl.BlockSpec((1, c), lambda i: (0, 0)),
            pl.BlockSpec((1, c), lambda i: (0, 0)),
            pl.BlockSpec((1, c), lambda i: (0, 0)),
            pl.BlockSpec((1, 2 * c), lambda i: (0, 0)),
            pl.BlockSpec((1, 2 * c), lambda i: (0, 0)),
            pl.BlockSpec((4 * c, 4 * c), lambda i: (0, 0)),
        ],
        out_specs=out_specs,
        out_shape=out_shape,
    )(gath, ae, u, dist3d, wd, enc_sc, enc_sh, dg_sc, dg_sh, watt)


def _s6b_body(tp_ref, tr_ref, p8_ref, ps_ref, ph_ref, rs_ref, rh_ref,
              adg_ref, cdg_ref, bdg_ref, aen_ref, cen_ref, ben_ref,
              rec_out, ae2_out, u2_out, tab_out):
    h2 = _lrelu(tp_ref[...] * ps_ref[...] + ph_ref[...])
    rec_out[...] = _lrelu(tr_ref[...] * rs_ref[...] + rh_ref[...])
    p8 = p8_ref[...]
    ae2_out[...] = jnp.dot(p8, aen_ref[...], preferred_element_type=f32) + ben_ref[...]
    u2_out[...] = jnp.dot(h2, adg_ref[...], preferred_element_type=f32) + bdg_ref[...]
    ce2 = jnp.dot(p8, cen_ref[...], preferred_element_type=f32)
    w2 = jnp.dot(h2, cdg_ref[...], preferred_element_type=f32)
    tab_out[...] = jnp.concatenate([h2, ce2, w2], axis=1)


def _s6b(t_post, t_raw, pos8, consts):
    vec = lambda c: pl.BlockSpec((1, c), lambda i: (0, 0))
    mat = lambda a, b: pl.BlockSpec((a, b), lambda i: (0, 0))
    return pl.pallas_call(
        _s6b_body,
        grid=(NRT,),
        in_specs=[
            pl.BlockSpec((RT, 32), lambda i: (i, 0)),
            pl.BlockSpec((RT, 128), lambda i: (i, 0)),
            pl.BlockSpec((RT, 8), lambda i: (i, 0)),
            vec(32), vec(32), vec(128), vec(128),
            mat(32, 64), mat(32, 64), vec(64),
            mat(8, 32), mat(8, 32), vec(32),
        ],
        out_specs=[
            pl.BlockSpec((RT, 128), lambda i: (i, 0)),
            pl.BlockSpec((RT, 32), lambda i: (i, 0)),
            pl.BlockSpec((RT, 64), lambda i: (i, 0)),
            pl.BlockSpec((RT, 128), lambda i: (i, 0)),
        ],
        out_shape=[
            jax.ShapeDtypeStruct((N, 128), f32),
            jax.ShapeDtypeStruct((N, 32), f32),
            jax.ShapeDtypeStruct((N, 64), f32),
            jax.ShapeDtypeStruct((N, 128), f32),
        ],
    )(t_post, t_raw, pos8, *consts)


def _s10c_body(tm_ref, sc_ref, s_ref, h_ref, out_ref):
    out_ref[...] = _lrelu(tm_ref[...] * s_ref[...] + h_ref[...] + sc_ref[...])


def _s10c(t_m, sc, scale, shift):
    return pl.pallas_call(
        _s10c_body,
        grid=(NRT,),
        in_specs=[
            pl.BlockSpec((RT, 128), lambda i: (i, 0)),
            pl.BlockSpec((RT, 128), lambda i: (i, 0)),
            pl.BlockSpec((1, 128), lambda i: (0, 0)),
            pl.BlockSpec((1, 128), lambda i: (0, 0)),
        ],
        out_specs=pl.BlockSpec((RT, 128), lambda i: (i, 0)),
        out_shape=jax.ShapeDtypeStruct((N, 128), f32),
    )(t_m, sc, scale, shift)


def _split3(w, c):
    a = w[0:c] - w[2 * c:3 * c]
    cc = w[c:2 * c] + w[2 * c:3 * c]
    return a, cc


def _split_enc(w):
    a = w[0:3] - w[6:9]
    cc = w[3:6] + w[6:9]
    pad = lambda m: jnp.concatenate([m, jnp.zeros((5, m.shape[1]), f32)], axis=0)
    return pad(a), pad(cc), w[9:10]


def kernel(pos, x, params):
    B = pos.shape[0]
    p = params
    pos2d = pos.reshape(N, 3)
    x2d = x.reshape(N, 128)
    pos8 = jnp.concatenate([pos2d, jnp.zeros((N, 5), f32)], axis=1)
    pos_pad = jnp.concatenate(
        [pos8, jnp.full((NPAD - N, 8), 1e3, f32)], axis=0)
    q8T = jnp.concatenate(
        [pos2d.T, jnp.full((3, NPAD - N), 1e3, f32)], axis=1)
    q8T = jnp.concatenate([q8T, jnp.zeros((5, NPAD), f32)], axis=0)

    nbrT = _knn(pos_pad, q8T)
    idx_flat = nbrT[:, :N].T.reshape(E)
    idx3 = jnp.concatenate(
        [idx_flat, jnp.zeros((E_PAD - E,), i32)]).reshape(NW, CH_PER_W, CHUNK)

    row = lambda v: v[None, :]

    (t_sc, s_sc, q_sc), (t1, s1, q1) = _linstat([
        (x2d, None, None, p["sc"]["W"], row(p["sc"]["b"])),
        (x2d, None, None, p["mlp1"]["W"], row(p["mlp1"]["b"])),
    ])
    sc_scale, sc_shift = _bn_fold(s_sc, q_sc, N, p["sc"]["g"], p["sc"]["be"])
    h1_scale, h1_shift = _bn_fold(s1, q1, N, p["mlp1"]["g"], p["mlp1"]["be"])

    l1, l2 = p["l1"], p["l2"]
    a_dg1, c_dg1 = _split3(l1["dg"]["W"], 16)
    a_en1, c_en1, wd1 = _split_enc(l1["enc"]["W"])
    sc_arr, ae1, u1, tab1 = _s2b(
        t_sc, t1, pos8,
        [sc_scale, sc_shift, h1_scale, h1_shift,
         a_dg1, c_dg1, row(l1["dg"]["b"]),
         a_en1, c_en1, row(l1["enc"]["b"])])

    gath1 = _sc_gather_rows(tab1, idx3, 128)
    se1, qe1, sd1, qd1, dist3d = _edge_stats(
        gath1, ae1, u1, pos8, wd1, 16, 128, True)
    enc1_sc, enc1_sh = _bn_fold(se1, qe1, E, l1["enc"]["g"], l1["enc"]["be"])
    dg1_sc, dg1_sh = _bn_fold(sd1, qd1, E, l1["dg"]["g"], l1["dg"]["be"])
    att1, raw1 = _edge_aggr(gath1, ae1, u1, dist3d, wd1, enc1_sc, enc1_sh,
                            dg1_sc, dg1_sh, l1["att"]["W"], 16, 128, True)

    (t_p1, s_p1, q_p1), (t_r1, s_r1, q_r1) = _linstat([
        (att1, None, None, l1["post"]["W"], row(l1["post"]["b"])),
        (raw1, None, None, l1["raw"]["W"], row(l1["raw"]["b"])),
    ])
    p1_scale, p1_shift = _bn_fold(s_p1, q_p1, N, l1["post"]["g"], l1["post"]["be"])
    r1_scale, r1_shift = _bn_fold(s_r1, q_r1, N, l1["raw"]["g"], l1["raw"]["be"])

    a_dg2, c_dg2 = _split3(l2["dg"]["W"], 32)
    a_en2, c_en2, wd2 = _split_enc(l2["enc"]["W"])
    rec, ae2, u2, tab2 = _s6b(
        t_p1, t_r1, pos8,
        [p1_scale, p1_shift, r1_scale, r1_shift,
         a_dg2, c_dg2, row(l2["dg"]["b"]),
         a_en2, c_en2, row(l2["enc"]["b"])])

    gath2 = _sc_gather_rows(tab2, idx3, 128)
    se2, qe2, sd2, qd2 = _edge_stats(
        gath2, ae2, u2, dist3d, wd2, 32, 128, False)
    enc2_sc, enc2_sh = _bn_fold(se2, qe2, E, l2["enc"]["g"], l2["enc"]["be"])
    dg2_sc, dg2_sh = _bn_fold(sd2, qd2, E, l2["dg"]["g"], l2["dg"]["be"])
    att2 = _edge_aggr(gath2, ae2, u2, dist3d, wd2, enc2_sc, enc2_sh,
                      dg2_sc, dg2_sh, l2["att"]["W"], 32, 128, False)[0]

    [(t_p2, s_p2, q_p2)] = _linstat([
        (att2, None, None, l2["post"]["W"], row(l2["post"]["b"])),
    ])
    p2_scale, p2_shift = _bn_fold(s_p2, q_p2, N, l2["post"]["g"], l2["post"]["be"])

    [(t_m, s_m, q_m)] = _linstat([
        (t_p2, p2_scale, p2_shift, p["mlp2"]["W"], row(p["mlp2"]["b"])),
    ])
    m_scale, m_shift = _bn_fold(s_m, q_m, N, p["mlp2"]["g"], p["mlp2"]["be"])

    out = _s10c(t_m, sc_arr, m_scale, m_shift)

    return (out.reshape(B, N, 128), pos2d.reshape(B, N, 3),
            rec.reshape(B, N, 128))
